# baseline (speedup 1.0000x reference)
import functools

import jax
import jax.numpy as jnp
from jax import lax
from jax.experimental import pallas as pl
from jax.experimental.pallas import tpu as pltpu
from jax.experimental.pallas import tpu_sc as plsc

N = 10000
D = 128
EPS = 1e-5

NC = 2
NS = 16
NW = NC * NS
CHUNK = 128
SC0_FRAC = 0.68
NP = 10240
ROWS_PER_TILE = NP // NS
DEGW = 128

BLK = 512
GRID = NP // BLK

_F32 = jnp.float32


def _phases(c0):
    ph = -(-(-(-c0 // 3)) // 8) * 8
    offs = list(range(0, c0, ph))
    return ph, [(o, min(ph, c0 - o)) for o in offs]


def _make_sc_agg(C0, C1):
    mesh = plsc.VectorSubcoreMesh(core_axis_name="c", subcore_axis_name="s")
    ph, phases = _phases(C0)

    def body(m_hbm, src_hbm, dst_hbm, acc_out, src_v, dst_v, rows_v,
             zrow_v, acc_sh, gsem):
        cid = lax.axis_index("c")
        sid = lax.axis_index("s")
        wid = cid * NS + sid
        cnt = lax.select(cid == 0, jnp.int32(C0), jnp.int32(C1))

        def zfill(i, _):
            r = i // (D // 16)
            c = i % (D // 16)
            zrow_v[r, pl.ds(c * 16, 16)] = jnp.zeros((16,), _F32)
            return 0
        lax.fori_loop(0, 16 * (D // 16), zfill, 0)

        base = sid * ROWS_PER_TILE
        def zcopy(i, _):
            pltpu.sync_copy(zrow_v, acc_sh.at[pl.ds(base + i * 16, 16)])
            return 0
        lax.fori_loop(0, ROWS_PER_TILE // 16, zcopy, 0)

        plsc.subcore_barrier()

        for off, cap in phases:
            n = jnp.clip(cnt - off, 0, cap)
            pltpu.sync_copy(src_hbm.at[wid, pl.ds(off, cap)],
                            src_v.at[pl.ds(0, cap)])
            pltpu.sync_copy(dst_hbm.at[wid, pl.ds(off, cap)],
                            dst_v.at[pl.ds(0, cap)])
            for b in range(2):
                @pl.when(b < n)
                def _():
                    pltpu.async_copy(m_hbm.at[src_v.at[b]], rows_v.at[b],
                                     gsem)

            def chunk(j, _):
                b = j % 2
                pltpu.make_async_copy(m_hbm.at[src_v.at[j]],
                                      rows_v.at[b], gsem).wait()
                pltpu.sync_copy(rows_v.at[b], acc_sh.at[dst_v.at[j]],
                                add=True)

                @pl.when(j + 2 < n)
                def _():
                    pltpu.async_copy(m_hbm.at[src_v.at[j + 2]],
                                     rows_v.at[b], gsem)
                return 0
            lax.fori_loop(0, n, chunk, 0)

        plsc.subcore_barrier()

        pltpu.sync_copy(acc_sh.at[pl.ds(base, ROWS_PER_TILE)],
                        acc_out.at[cid, pl.ds(base, ROWS_PER_TILE)])

    return pl.kernel(
        body,
        out_type=[jax.ShapeDtypeStruct((NC, NP, D), _F32)],
        mesh=mesh,
        scratch_types=[
            pltpu.VMEM((ph, CHUNK), jnp.int32),
            pltpu.VMEM((ph, CHUNK), jnp.int32),
            pltpu.VMEM((2, CHUNK, D), _F32),
            pltpu.VMEM((16, D), _F32),
            pltpu.VMEM_SHARED((NP, D), _F32),
            pltpu.SemaphoreType.DMA,
        ])


def _make_sc_deg(C0, C1):
    mesh = plsc.VectorSubcoreMesh(core_axis_name="c", subcore_axis_name="s")
    ph, phases = _phases(C0)

    def body(dst_hbm, deg_out, dst_v, ones_v, zdeg_v, deg_sh):
        cid = lax.axis_index("c")
        sid = lax.axis_index("s")
        wid = cid * NS + sid
        cnt = lax.select(cid == 0, jnp.int32(C0), jnp.int32(C1))

        def ofill(i, _):
            ones_v[i, pl.ds(0, DEGW)] = jnp.ones((DEGW,), _F32)
            zdeg_v[i % 16, pl.ds(0, DEGW)] = jnp.zeros((DEGW,), _F32)
            return 0
        lax.fori_loop(0, CHUNK, ofill, 0)

        base = sid * ROWS_PER_TILE
        def zcopy(i, _):
            pltpu.sync_copy(zdeg_v, deg_sh.at[pl.ds(base + i * 16, 16)])
            return 0
        lax.fori_loop(0, ROWS_PER_TILE // 16, zcopy, 0)

        plsc.subcore_barrier()

        for off, cap in phases:
            n = jnp.clip(cnt - off, 0, cap)
            pltpu.sync_copy(dst_hbm.at[wid, pl.ds(off, cap)],
                            dst_v.at[pl.ds(0, cap)])

            def chunk(j, _):
                pltpu.sync_copy(ones_v, deg_sh.at[dst_v.at[j]], add=True)
                return 0
            lax.fori_loop(0, n, chunk, 0)

        plsc.subcore_barrier()
        pltpu.sync_copy(deg_sh.at[pl.ds(base, ROWS_PER_TILE)],
                        deg_out.at[cid, pl.ds(base, ROWS_PER_TILE)])

    return pl.kernel(
        body,
        out_type=[jax.ShapeDtypeStruct((NC, NP, DEGW), _F32)],
        mesh=mesh,
        scratch_types=[
            pltpu.VMEM((ph, CHUNK), jnp.int32),
            pltpu.VMEM((CHUNK, DEGW), _F32),
            pltpu.VMEM((16, DEGW), _F32),
            pltpu.VMEM_SHARED((NP, DEGW), _F32),
        ])


def _dotT(a, w):
    return lax.dot_general(a, w, (((1,), (1,)), ((), ())),
                           preferred_element_type=_F32)


def _row_spec():
    return pl.BlockSpec((BLK, D), lambda i: (i, 0))


def _deg_spec():
    return pl.BlockSpec((BLK, DEGW), lambda i: (i, 0))


def _full_spec(shape):
    return pl.BlockSpec(shape, lambda i: tuple(0 for _ in shape))


def _tc_in(x, Wl, Wr, bl):
    def body(x_ref, wl_ref, wr_ref, bl_ref, m_ref, z_ref):
        xv = x_ref[...]
        m_ref[...] = _dotT(xv, wl_ref[...])
        z_ref[...] = _dotT(xv, wr_ref[...]) + bl_ref[...]
    return pl.pallas_call(
        body,
        grid=(GRID,),
        in_specs=[_row_spec(), _full_spec((D, D)), _full_spec((D, D)),
                  _full_spec((1, D))],
        out_specs=[_row_spec(), _row_spec()],
        out_shape=[jax.ShapeDtypeStruct((NP, D), _F32),
                   jax.ShapeDtypeStruct((NP, D), _F32)],
    )(x, Wl, Wr, bl)


def _tc_mid(acc0, acc1, deg0, deg1, z, g, be, rm, rv, Wl, Wr, bl):
    def body(a0, a1, d0, d1, z_ref, g_ref, be_ref, rm_ref, rv_ref,
             wl_ref, wr_ref, bl_ref, m_ref, z2_ref):
        deg = d0[:, 0:1] + d1[:, 0:1]
        inv = 1.0 / jnp.maximum(deg, 1.0)
        s = (a0[...] + a1[...]) * inv + z_ref[...]
        scale = g_ref[...] * lax.rsqrt(rv_ref[...] + EPS)
        h = jnp.maximum((s - rm_ref[...]) * scale + be_ref[...], 0.0)
        m_ref[...] = _dotT(h, wl_ref[...])
        z2_ref[...] = _dotT(h, wr_ref[...]) + bl_ref[...]
    return pl.pallas_call(
        body,
        grid=(GRID,),
        in_specs=[_row_spec(), _row_spec(), _deg_spec(), _deg_spec(),
                  _row_spec(),
                  _full_spec((1, D)), _full_spec((1, D)), _full_spec((1, D)),
                  _full_spec((1, D)),
                  _full_spec((D, D)), _full_spec((D, D)), _full_spec((1, D))],
        out_specs=[_row_spec(), _row_spec()],
        out_shape=[jax.ShapeDtypeStruct((NP, D), _F32),
                   jax.ShapeDtypeStruct((NP, D), _F32)],
    )(acc0, acc1, deg0, deg1, z, g, be, rm, rv, Wl, Wr, bl)


def _tc_out(acc0, acc1, deg0, deg1, z):
    def body(a0, a1, d0, d1, z_ref, o_ref):
        deg = d0[:, 0:1] + d1[:, 0:1]
        inv = 1.0 / jnp.maximum(deg, 1.0)
        o_ref[...] = (a0[...] + a1[...]) * inv + z_ref[...]
    return pl.pallas_call(
        body,
        grid=(GRID,),
        in_specs=[_row_spec(), _row_spec(), _deg_spec(), _deg_spec(),
                  _row_spec()],
        out_specs=_row_spec(),
        out_shape=jax.ShapeDtypeStruct((NP, D), _F32),
    )(acc0, acc1, deg0, deg1, z)


def kernel(x, edge_index, Wl0, bl0, Wr0, g0, be0, rm0, rv0,
           Wl1, bl1, Wr1, g1, be1, rm1, rv1, Wl2, bl2, Wr2):
    E = edge_index.shape[1]
    T = -(-E // (NS * CHUNK))
    C0 = min(max(int(round(T * SC0_FRAC)), 1), T)
    C1 = T - C0 + 1
    cap0 = NS * C0 * CHUNK

    src = edge_index[0]
    dst = edge_index[1]
    def _layout(v, padval):
        a = v[:cap0].reshape(NS, C0, CHUNK)
        rem = v[cap0:]
        e_pad = NS * C1 * CHUNK - rem.shape[0]
        b = jnp.concatenate([rem, jnp.full((e_pad,), padval, jnp.int32)])
        b = b.reshape(NS, C1, CHUNK)
        b = jnp.pad(b, ((0, 0), (0, C0 - C1), (0, 0)),
                    constant_values=padval)
        return jnp.concatenate([a, b], axis=0)

    src_p = _layout(src, 0)
    dst_p = _layout(dst, N)

    x_pad = jnp.concatenate([x, jnp.zeros((NP - N, D), _F32)])

    r1 = lambda v: v.reshape(1, D)

    sc_agg = _make_sc_agg(C0, C1)
    sc_deg = _make_sc_deg(C0, C1)

    m0, z0 = _tc_in(x_pad, Wl0, Wr0, r1(bl0))
    (deg,) = sc_deg(dst_p)
    (acc,) = sc_agg(m0, src_p, dst_p)
    m1, z1 = _tc_mid(acc[0], acc[1], deg[0], deg[1], z0,
                     r1(g0), r1(be0), r1(rm0), r1(rv0), Wl1, Wr1, r1(bl1))
    (acc,) = sc_agg(m1, src_p, dst_p)
    m2, z2 = _tc_mid(acc[0], acc[1], deg[0], deg[1], z1,
                     r1(g1), r1(be1), r1(rm1), r1(rv1), Wl2, Wr2, r1(bl2))
    (acc,) = sc_agg(m2, src_p, dst_p)
    out = _tc_out(acc[0], acc[1], deg[0], deg[1], z2)
    return out[:N]


# speedup vs baseline: 1.1787x; 1.1787x over previous
import jax
import jax.numpy as jnp
from jax import lax
from jax.experimental import pallas as pl
from jax.experimental.pallas import tpu as pltpu
from jax.experimental.pallas import tpu_sc as plsc

N = 10000
D = 128
EPS = 1e-5

NC = 2
NS = 16
CHUNK = 128
NP = 10240
ROWS_PER_TILE = NP // NS

BLK = 512
GRID = NP // BLK

_F32 = jnp.float32


def _phases(c0):
    ph = -(-(-(-c0 // 4)) // 8) * 8
    offs = list(range(0, c0, ph))
    return ph, [(o, min(ph, c0 - o)) for o in offs]


def _make_sc_agg(C, with_deg):
    mesh = plsc.VectorSubcoreMesh(core_axis_name="c", subcore_axis_name="s")
    ph, phases = _phases(C)
    out_type = [jax.ShapeDtypeStruct((NP, D), _F32)]
    if with_deg:
        out_type.append(jax.ShapeDtypeStruct((NP, D), _F32))

    def body(m_hbm, src_hbm, dst_hbm, acc_out, *rest):
        if with_deg:
            deg_out, src_v, dst_v, rows_v, zrow_v, acc_sh, gsem = rest
        else:
            src_v, dst_v, rows_v, zrow_v, acc_sh, gsem = rest
        cid = lax.axis_index("c")
        sid = lax.axis_index("s")
        is_gather = cid == 0
        active = (cid <= (1 if with_deg else 0))

        def zfill(i, _):
            r = i // (D // 16)
            c = i % (D // 16)
            zrow_v[r, pl.ds(c * 16, 16)] = jnp.zeros((16,), _F32)
            return 0
        lax.fori_loop(0, 16 * (D // 16), zfill, 0)

        if with_deg:
            @pl.when(~is_gather)
            def _():
                def ofill(i, _):
                    r = i // (D // 16)
                    c = i % (D // 16)
                    rows_v[0, r, pl.ds(c * 16, 16)] = jnp.ones((16,), _F32)
                    return 0
                lax.fori_loop(0, CHUNK * (D // 16), ofill, 0)

        base = sid * ROWS_PER_TILE

        @pl.when(active)
        def _():
            def zcopy(i, _):
                pltpu.sync_copy(zrow_v, acc_sh.at[pl.ds(base + i * 16, 16)])
                return 0
            lax.fori_loop(0, ROWS_PER_TILE // 16, zcopy, 0)

        plsc.subcore_barrier()

        for off, cap in phases:
            @pl.when(is_gather)
            def _():
                pltpu.sync_copy(src_hbm.at[sid, pl.ds(off, cap)],
                                src_v.at[pl.ds(0, cap)])

            @pl.when(active)
            def _():
                pltpu.sync_copy(dst_hbm.at[sid, pl.ds(off, cap)],
                                dst_v.at[pl.ds(0, cap)])

            @pl.when(is_gather)
            def _():
                for b in range(2):
                    pltpu.async_copy(m_hbm.at[src_v.at[b]],
                                     rows_v.at[b], gsem)

                def chunk(j, _):
                    b = j % 2
                    pltpu.make_async_copy(m_hbm.at[src_v.at[j]],
                                          rows_v.at[b], gsem).wait()
                    pltpu.sync_copy(rows_v.at[b], acc_sh.at[dst_v.at[j]],
                                    add=True)

                    @pl.when(j + 2 < cap)
                    def _():
                        pltpu.async_copy(m_hbm.at[src_v.at[j + 2]],
                                         rows_v.at[b], gsem)
                    return 0
                lax.fori_loop(0, cap, chunk, 0)

            if with_deg:
                @pl.when(~is_gather)
                def _():
                    def dchunk(j, _):
                        pltpu.sync_copy(rows_v.at[0],
                                        acc_sh.at[dst_v.at[j]], add=True)
                        return 0
                    lax.fori_loop(0, cap, dchunk, 0)

        plsc.subcore_barrier()

        @pl.when(is_gather)
        def _():
            pltpu.sync_copy(acc_sh.at[pl.ds(base, ROWS_PER_TILE)],
                            acc_out.at[pl.ds(base, ROWS_PER_TILE)])
        if with_deg:
            @pl.when(~is_gather)
            def _():
                pltpu.sync_copy(acc_sh.at[pl.ds(base, ROWS_PER_TILE)],
                                deg_out.at[pl.ds(base, ROWS_PER_TILE)])

    return pl.kernel(
        body,
        out_type=out_type,
        mesh=mesh,
        scratch_types=[
            pltpu.VMEM((ph, CHUNK), jnp.int32),
            pltpu.VMEM((ph, CHUNK), jnp.int32),
            pltpu.VMEM((2, CHUNK, D), _F32),
            pltpu.VMEM((16, D), _F32),
            pltpu.VMEM_SHARED((NP, D), _F32),
            pltpu.SemaphoreType.DMA,
        ])


def _dotT(a, w):
    return lax.dot_general(a, w, (((1,), (1,)), ((), ())),
                           preferred_element_type=_F32)


def _row_spec():
    return pl.BlockSpec((BLK, D), lambda i: (i, 0))


def _full_spec(shape):
    return pl.BlockSpec(shape, lambda i: tuple(0 for _ in shape))


def _tc_in(x, Wl, Wr, bl):
    def body(x_ref, wl_ref, wr_ref, bl_ref, m_ref, z_ref):
        xv = x_ref[...]
        m_ref[...] = _dotT(xv, wl_ref[...])
        z_ref[...] = _dotT(xv, wr_ref[...]) + bl_ref[...]
    return pl.pallas_call(
        body,
        grid=(GRID,),
        in_specs=[_row_spec(), _full_spec((D, D)), _full_spec((D, D)),
                  _full_spec((1, D))],
        out_specs=[_row_spec(), _row_spec()],
        out_shape=[jax.ShapeDtypeStruct((NP, D), _F32),
                   jax.ShapeDtypeStruct((NP, D), _F32)],
    )(x, Wl, Wr, bl)


def _tc_mid(acc, deg, z, g, be, rm, rv, Wl, Wr, bl):
    def body(a_ref, d_ref, z_ref, g_ref, be_ref, rm_ref, rv_ref,
             wl_ref, wr_ref, bl_ref, m_ref, z2_ref):
        inv = 1.0 / jnp.maximum(d_ref[:, 0:1], 1.0)
        s = a_ref[...] * inv + z_ref[...]
        scale = g_ref[...] * lax.rsqrt(rv_ref[...] + EPS)
        h = jnp.maximum((s - rm_ref[...]) * scale + be_ref[...], 0.0)
        m_ref[...] = _dotT(h, wl_ref[...])
        z2_ref[...] = _dotT(h, wr_ref[...]) + bl_ref[...]
    return pl.pallas_call(
        body,
        grid=(GRID,),
        in_specs=[_row_spec(), _row_spec(), _row_spec(),
                  _full_spec((1, D)), _full_spec((1, D)), _full_spec((1, D)),
                  _full_spec((1, D)),
                  _full_spec((D, D)), _full_spec((D, D)), _full_spec((1, D))],
        out_specs=[_row_spec(), _row_spec()],
        out_shape=[jax.ShapeDtypeStruct((NP, D), _F32),
                   jax.ShapeDtypeStruct((NP, D), _F32)],
    )(acc, deg, z, g, be, rm, rv, Wl, Wr, bl)


def _tc_out(acc, deg, z):
    def body(a_ref, d_ref, z_ref, o_ref):
        inv = 1.0 / jnp.maximum(d_ref[:, 0:1], 1.0)
        o_ref[...] = a_ref[...] * inv + z_ref[...]
    return pl.pallas_call(
        body,
        grid=(GRID,),
        in_specs=[_row_spec(), _row_spec(), _row_spec()],
        out_specs=_row_spec(),
        out_shape=jax.ShapeDtypeStruct((NP, D), _F32),
    )(acc, deg, z)


def kernel(x, edge_index, Wl0, bl0, Wr0, g0, be0, rm0, rv0,
           Wl1, bl1, Wr1, g1, be1, rm1, rv1, Wl2, bl2, Wr2):
    E = edge_index.shape[1]
    C = -(-E // (NS * CHUNK))
    e_pad = NS * C * CHUNK - E

    src = edge_index[0]
    dst = edge_index[1]
    src_p = jnp.concatenate([src, jnp.zeros((e_pad,), jnp.int32)])
    dst_p = jnp.concatenate([dst, jnp.full((e_pad,), N, jnp.int32)])
    src_p = src_p.reshape(NS, C, CHUNK)
    dst_p = dst_p.reshape(NS, C, CHUNK)

    x_pad = jnp.concatenate([x, jnp.zeros((NP - N, D), _F32)])

    r1 = lambda v: v.reshape(1, D)

    sc_agg_deg = _make_sc_agg(C, with_deg=True)
    sc_agg = _make_sc_agg(C, with_deg=False)

    m0, z0 = _tc_in(x_pad, Wl0, Wr0, r1(bl0))
    acc, deg = sc_agg_deg(m0, src_p, dst_p)
    m1, z1 = _tc_mid(acc, deg, z0,
                     r1(g0), r1(be0), r1(rm0), r1(rv0), Wl1, Wr1, r1(bl1))
    (acc,) = sc_agg(m1, src_p, dst_p)
    m2, z2 = _tc_mid(acc, deg, z1,
                     r1(g1), r1(be1), r1(rm1), r1(rv1), Wl2, Wr2, r1(bl2))
    (acc,) = sc_agg(m2, src_p, dst_p)
    out = _tc_out(acc, deg, z2)
    return out[:N]


# speedup vs baseline: 1.2006x; 1.0186x over previous
import jax
import jax.numpy as jnp
from jax import lax
from jax.experimental import pallas as pl
from jax.experimental.pallas import tpu as pltpu
from jax.experimental.pallas import tpu_sc as plsc

N = 10000
D = 128
EPS = 1e-5

NC = 2
NS = 16
CHUNK = 128
NP = 10240
ROWS_PER_TILE = NP // NS

BLK = 512
GRID = NP // BLK

_F32 = jnp.float32


DEGW = 16


def _phases(c0):
    ph = -(-(-(-c0 // 3)) // 8) * 8
    offs = list(range(0, c0, ph))
    return ph, [(o, min(ph, c0 - o)) for o in offs]


def _make_sc_agg(C, with_deg):
    mesh = plsc.VectorSubcoreMesh(core_axis_name="c", subcore_axis_name="s")
    ph, phases = _phases(C)
    out_type = [jax.ShapeDtypeStruct((NP, D), _F32)]
    if with_deg:
        out_type.append(jax.ShapeDtypeStruct((NP, D), _F32))

    def body(m_hbm, src_hbm, dst_hbm, acc_out, *rest):
        if with_deg:
            deg_out, src_v, dst_v, rows_v, acc_sh, gsem = rest
        else:
            src_v, dst_v, rows_v, acc_sh, gsem = rest
        cid = lax.axis_index("c")
        sid = lax.axis_index("s")
        is_gather = cid == 0
        active = (cid <= (1 if with_deg else 0))

        def zfill(i, _):
            r = i // (D // 16)
            c = i % (D // 16)
            rows_v[0, r, pl.ds(c * 16, 16)] = jnp.zeros((16,), _F32)
            return 0
        lax.fori_loop(0, CHUNK * (D // 16), zfill, 0)

        base = sid * ROWS_PER_TILE

        @pl.when(active)
        def _():
            def zcopy(i, _):
                pltpu.sync_copy(rows_v.at[0],
                                acc_sh.at[pl.ds(base + i * CHUNK, CHUNK)])
                return 0
            lax.fori_loop(0, ROWS_PER_TILE // CHUNK, zcopy, 0)

        if with_deg:
            @pl.when(~is_gather)
            def _():
                def ofill(i, _):
                    r = i // (D // 16)
                    c = i % (D // 16)
                    rows_v[0, r, pl.ds(c * 16, 16)] = jnp.ones((16,), _F32)
                    return 0
                lax.fori_loop(0, CHUNK * (D // 16), ofill, 0)

        plsc.subcore_barrier()

        for off, cap in phases:
            @pl.when(is_gather)
            def _():
                pltpu.sync_copy(src_hbm.at[sid, pl.ds(off, cap)],
                                src_v.at[pl.ds(0, cap)])

            @pl.when(active)
            def _():
                pltpu.sync_copy(dst_hbm.at[sid, pl.ds(off, cap)],
                                dst_v.at[pl.ds(0, cap)])

            @pl.when(is_gather)
            def _():
                for b in range(2):
                    pltpu.async_copy(m_hbm.at[src_v.at[b]],
                                     rows_v.at[b], gsem)

                def chunk(j, _):
                    b = j % 2
                    pltpu.make_async_copy(m_hbm.at[pl.ds(0, CHUNK)],
                                          rows_v.at[b], gsem).wait()
                    pltpu.sync_copy(rows_v.at[b], acc_sh.at[dst_v.at[j]],
                                    add=True)

                    @pl.when(j + 2 < cap)
                    def _():
                        pltpu.async_copy(m_hbm.at[src_v.at[j + 2]],
                                         rows_v.at[b], gsem)
                    return 0
                lax.fori_loop(0, cap, chunk, 0)

            if with_deg:
                @pl.when(~is_gather)
                def _():
                    def dchunk(j, _):
                        pltpu.sync_copy(rows_v.at[0],
                                        acc_sh.at[dst_v.at[j]], add=True)
                        return 0
                    lax.fori_loop(0, cap, dchunk, 0)

        plsc.subcore_barrier()

        @pl.when(is_gather)
        def _():
            pltpu.sync_copy(acc_sh.at[pl.ds(base, ROWS_PER_TILE)],
                            acc_out.at[pl.ds(base, ROWS_PER_TILE)])
        if with_deg:
            @pl.when(~is_gather)
            def _():
                pltpu.sync_copy(acc_sh.at[pl.ds(base, ROWS_PER_TILE)],
                                deg_out.at[pl.ds(base, ROWS_PER_TILE)])

    return pl.kernel(
        body,
        out_type=out_type,
        mesh=mesh,
        scratch_types=[
            pltpu.VMEM((ph, CHUNK), jnp.int32),
            pltpu.VMEM((ph, CHUNK), jnp.int32),
            pltpu.VMEM((2, CHUNK, D), _F32),
            pltpu.VMEM_SHARED((NP, D), _F32),
            pltpu.SemaphoreType.DMA,
        ])


def _dotT(a, w):
    return lax.dot_general(a, w, (((1,), (1,)), ((), ())),
                           preferred_element_type=_F32)


def _row_spec():
    return pl.BlockSpec((BLK, D), lambda i: (i, 0))


def _deg_spec():
    return pl.BlockSpec((BLK, D), lambda i: (i, 0))


def _full_spec(shape):
    return pl.BlockSpec(shape, lambda i: tuple(0 for _ in shape))


def _tc_in(x, Wl, Wr, bl):
    def body(x_ref, wl_ref, wr_ref, bl_ref, m_ref, z_ref):
        xv = x_ref[...]
        m_ref[...] = _dotT(xv, wl_ref[...])
        z_ref[...] = _dotT(xv, wr_ref[...]) + bl_ref[...]
    return pl.pallas_call(
        body,
        grid=(GRID,),
        in_specs=[_row_spec(), _full_spec((D, D)), _full_spec((D, D)),
                  _full_spec((1, D))],
        out_specs=[_row_spec(), _row_spec()],
        out_shape=[jax.ShapeDtypeStruct((NP, D), _F32),
                   jax.ShapeDtypeStruct((NP, D), _F32)],
    )(x, Wl, Wr, bl)


def _tc_mid(acc, deg, z, g, be, rm, rv, Wl, Wr, bl):
    def body(a_ref, d_ref, z_ref, g_ref, be_ref, rm_ref, rv_ref,
             wl_ref, wr_ref, bl_ref, m_ref, z2_ref):
        inv = 1.0 / jnp.maximum(d_ref[:, 0:1], 1.0)
        s = a_ref[...] * inv + z_ref[...]
        scale = g_ref[...] * lax.rsqrt(rv_ref[...] + EPS)
        h = jnp.maximum((s - rm_ref[...]) * scale + be_ref[...], 0.0)
        m_ref[...] = _dotT(h, wl_ref[...])
        z2_ref[...] = _dotT(h, wr_ref[...]) + bl_ref[...]
    return pl.pallas_call(
        body,
        grid=(GRID,),
        in_specs=[_row_spec(), _deg_spec(), _row_spec(),
                  _full_spec((1, D)), _full_spec((1, D)), _full_spec((1, D)),
                  _full_spec((1, D)),
                  _full_spec((D, D)), _full_spec((D, D)), _full_spec((1, D))],
        out_specs=[_row_spec(), _row_spec()],
        out_shape=[jax.ShapeDtypeStruct((NP, D), _F32),
                   jax.ShapeDtypeStruct((NP, D), _F32)],
    )(acc, deg, z, g, be, rm, rv, Wl, Wr, bl)


def _tc_out(acc, deg, z):
    def body(a_ref, d_ref, z_ref, o_ref):
        inv = 1.0 / jnp.maximum(d_ref[:, 0:1], 1.0)
        o_ref[...] = a_ref[...] * inv + z_ref[...]
    return pl.pallas_call(
        body,
        grid=(GRID,),
        in_specs=[_row_spec(), _deg_spec(), _row_spec()],
        out_specs=_row_spec(),
        out_shape=jax.ShapeDtypeStruct((NP, D), _F32),
    )(acc, deg, z)


def kernel(x, edge_index, Wl0, bl0, Wr0, g0, be0, rm0, rv0,
           Wl1, bl1, Wr1, g1, be1, rm1, rv1, Wl2, bl2, Wr2):
    E = edge_index.shape[1]
    C = -(-E // (NS * CHUNK))
    e_pad = NS * C * CHUNK - E

    src = edge_index[0]
    dst = edge_index[1]
    src_p = jnp.concatenate([src, jnp.zeros((e_pad,), jnp.int32)])
    dst_p = jnp.concatenate([dst, jnp.full((e_pad,), N, jnp.int32)])
    src_p = src_p.reshape(NS, C, CHUNK)
    dst_p = dst_p.reshape(NS, C, CHUNK)

    x_pad = jnp.concatenate([x, jnp.zeros((NP - N, D), _F32)])

    r1 = lambda v: v.reshape(1, D)

    sc_agg_deg = _make_sc_agg(C, with_deg=True)
    sc_agg = _make_sc_agg(C, with_deg=False)

    m0, z0 = _tc_in(x_pad, Wl0, Wr0, r1(bl0))
    acc, deg = sc_agg_deg(m0, src_p, dst_p)
    m1, z1 = _tc_mid(acc, deg, z0,
                     r1(g0), r1(be0), r1(rm0), r1(rv0), Wl1, Wr1, r1(bl1))
    (acc,) = sc_agg(m1, src_p, dst_p)
    m2, z2 = _tc_mid(acc, deg, z1,
                     r1(g1), r1(be1), r1(rm1), r1(rv1), Wl2, Wr2, r1(bl2))
    (acc,) = sc_agg(m2, src_p, dst_p)
    out = _tc_out(acc, deg, z2)
    return out[:N]
